# baseline (speedup 1.0000x reference)
import jax
import jax.numpy as jnp
from jax import lax
from jax.experimental import pallas as pl
from jax.experimental.pallas import tpu as pltpu
from jax.experimental.pallas import tpu_sc as plsc

N = 320000
D = 128
B = 1000
BP = 1024
NC = 2
NS = 16
NW = NC * NS
CH = 128
NCHUNKS = N // CH


def _seg_sum_body(x_hbm, b_hbm, z_hbm, out_hbm, xbuf, idxv, acc):
    cid = lax.axis_index("c")
    sid = lax.axis_index("s")
    wid = sid * NC + cid

    pltpu.sync_copy(z_hbm.at[pl.ds(sid * 64, 64)], acc.at[pl.ds(sid * 64, 64)])
    plsc.subcore_barrier()

    nchunks = (NCHUNKS - wid + NW - 1) // NW

    def chunk_body(k, _):
        chunk = wid + k * NW
        row0 = chunk * CH
        pltpu.sync_copy(x_hbm.at[pl.ds(row0, CH)], xbuf)
        pltpu.sync_copy(b_hbm.at[pl.ds(row0, CH)], idxv)
        pltpu.sync_copy(xbuf, acc.at[idxv], add=True)
        return 0

    lax.fori_loop(0, nchunks, chunk_body, 0)
    plsc.subcore_barrier()
    pltpu.sync_copy(acc.at[pl.ds(sid * 64, 64)],
                    out_hbm.at[pl.ds(cid * BP + sid * 64, 64)])


def _gate_pool_body(x_hbm, b_hbm, c_hbm, z_hbm, out_hbm, xbuf, cbuf, ybuf,
                    idxv, acc):
    cid = lax.axis_index("c")
    sid = lax.axis_index("s")
    wid = sid * NC + cid

    pltpu.sync_copy(z_hbm.at[pl.ds(sid * 64, 64)], acc.at[pl.ds(sid * 64, 64)])
    plsc.subcore_barrier()

    nchunks = (NCHUNKS - wid + NW - 1) // NW

    def chunk_body(k, _):
        chunk = wid + k * NW
        row0 = chunk * CH
        pltpu.sync_copy(x_hbm.at[pl.ds(row0, CH)], xbuf)
        pltpu.sync_copy(b_hbm.at[pl.ds(row0, CH)], idxv)
        pltpu.sync_copy(c_hbm.at[idxv], cbuf)

        lanes = lax.iota(jnp.int32, 16)
        dnums = lax.GatherDimensionNumbers(
            offset_dims=(), collapsed_slice_dims=(0,), start_index_map=(0,))

        @plsc.parallel_loop(0, CH, 1, unroll=8)
        def row_body(r):
            xv = [xbuf[r, pl.ds(16 * j, 16)] for j in range(8)]
            cv = [cbuf[r, pl.ds(16 * j, 16)] for j in range(8)]
            p = [xv[j] * cv[j] for j in range(8)]
            p = [p[0] + p[1], p[2] + p[3], p[4] + p[5], p[6] + p[7]]
            t = (p[0] + p[1]) + (p[2] + p[3])
            for k in (8, 4, 2, 1):
                t = t + lax.gather(
                    t, (lanes ^ k)[:, None], dimension_numbers=dnums,
                    slice_sizes=(1,),
                    mode=lax.GatherScatterMode.PROMISE_IN_BOUNDS)
            a16 = 1.0 / (1.0 + jnp.exp(-t))
            for j in range(8):
                ybuf[r, pl.ds(16 * j, 16)] = xv[j] * a16
        pltpu.sync_copy(ybuf, acc.at[idxv], add=True)
        return 0

    lax.fori_loop(0, nchunks, chunk_body, 0)
    plsc.subcore_barrier()
    pltpu.sync_copy(acc.at[pl.ds(sid * 64, 64)],
                    out_hbm.at[pl.ds(cid * BP + sid * 64, 64)])


def _gate_tc(p_ref, cnt_ref, w_ref, c_ref):
    s = p_ref[0:BP, :] + p_ref[BP:2 * BP, :]
    mean = s / cnt_ref[...]
    c_ref[...] = jnp.tanh(jnp.dot(mean, w_ref[...],
                                  preferred_element_type=jnp.float32))


def _combine_tc(q_ref, o_ref):
    o_ref[...] = q_ref[0:B, :] + q_ref[BP:BP + B, :]


def kernel(x, W, batch, c_size):
    batch = batch.astype(jnp.int32)
    zeros = jnp.zeros((BP, D), jnp.float32)
    cnt = jnp.maximum(c_size, 1).astype(jnp.float32)
    cnt = jnp.concatenate([cnt, jnp.ones((BP - B,), jnp.float32)])[:, None]

    mesh = plsc.VectorSubcoreMesh(core_axis_name="c", subcore_axis_name="s")

    seg_partial = pl.kernel(
        _seg_sum_body,
        out_type=jax.ShapeDtypeStruct((NC * BP, D), jnp.float32),
        mesh=mesh,
        scratch_types=[
            pltpu.VMEM((CH, D), jnp.float32),
            pltpu.VMEM((CH,), jnp.int32),
            pltpu.VMEM_SHARED((BP, D), jnp.float32),
        ],
    )(x, batch, zeros)

    c = pl.pallas_call(
        _gate_tc,
        out_shape=jax.ShapeDtypeStruct((BP, D), jnp.float32),
    )(seg_partial, cnt, W)

    out_partial = pl.kernel(
        _gate_pool_body,
        out_type=jax.ShapeDtypeStruct((NC * BP, D), jnp.float32),
        mesh=mesh,
        scratch_types=[
            pltpu.VMEM((CH, D), jnp.float32),
            pltpu.VMEM((CH, D), jnp.float32),
            pltpu.VMEM((CH, D), jnp.float32),
            pltpu.VMEM((CH,), jnp.int32),
            pltpu.VMEM_SHARED((BP, D), jnp.float32),
        ],
    )(x, batch, c, zeros)

    out = pl.pallas_call(
        _combine_tc,
        out_shape=jax.ShapeDtypeStruct((B, D), jnp.float32),
    )(out_partial)
    return out


# speedup vs baseline: 2.7462x; 2.7462x over previous
import jax
import jax.numpy as jnp
from jax import lax
from jax.experimental import pallas as pl
from jax.experimental.pallas import tpu as pltpu
from jax.experimental.pallas import tpu_sc as plsc

N = 320000
D = 128
B = 1000
BP = 1024
NC = 2
NS = 16
NW = NC * NS
CH = 128
NCHUNKS = N // CH


def _seg_sum_body(x_hbm, b_hbm, z_hbm, out_hbm, xbuf, idxv, acc):
    cid = lax.axis_index("c")
    sid = lax.axis_index("s")
    wid = sid * NC + cid

    pltpu.sync_copy(z_hbm.at[pl.ds(sid * 64, 64)], acc.at[pl.ds(sid * 64, 64)])
    plsc.subcore_barrier()

    nchunks = (NCHUNKS - wid + NW - 1) // NW

    def chunk_body(k, _):
        chunk = wid + k * NW
        row0 = chunk * CH
        pltpu.sync_copy(x_hbm.at[pl.ds(row0, CH)], xbuf)
        pltpu.sync_copy(b_hbm.at[pl.ds(row0, CH)], idxv)
        pltpu.sync_copy(xbuf, acc.at[idxv], add=True)
        return 0

    lax.fori_loop(0, nchunks, chunk_body, 0)
    plsc.subcore_barrier()
    pltpu.sync_copy(acc.at[pl.ds(sid * 64, 64)],
                    out_hbm.at[pl.ds(cid * BP + sid * 64, 64)])


def _gate_pool_body(x_hbm, b_hbm, c_hbm, z_hbm, out_hbm, xbuf, cbuf, ybuf,
                    idxv, acc, c_sh):
    cid = lax.axis_index("c")
    sid = lax.axis_index("s")
    wid = sid * NC + cid

    pltpu.sync_copy(z_hbm.at[pl.ds(sid * 64, 64)], acc.at[pl.ds(sid * 64, 64)])
    pltpu.sync_copy(c_hbm.at[pl.ds(sid * 64, 64)], c_sh.at[pl.ds(sid * 64, 64)])
    plsc.subcore_barrier()

    nchunks = (NCHUNKS - wid + NW - 1) // NW

    def chunk_body(k, _):
        chunk = wid + k * NW
        row0 = chunk * CH
        pltpu.sync_copy(x_hbm.at[pl.ds(row0, CH)], xbuf)
        pltpu.sync_copy(b_hbm.at[pl.ds(row0, CH)], idxv)
        pltpu.sync_copy(c_sh.at[idxv], cbuf)

        lanes = lax.iota(jnp.int32, 16)
        dnums = lax.GatherDimensionNumbers(
            offset_dims=(), collapsed_slice_dims=(0,), start_index_map=(0,))

        @plsc.parallel_loop(0, CH, 1, unroll=8)
        def row_body(r):
            xv = [xbuf[r, pl.ds(16 * j, 16)] for j in range(8)]
            cv = [cbuf[r, pl.ds(16 * j, 16)] for j in range(8)]
            p = [xv[j] * cv[j] for j in range(8)]
            p = [p[0] + p[1], p[2] + p[3], p[4] + p[5], p[6] + p[7]]
            t = (p[0] + p[1]) + (p[2] + p[3])
            for m in (8, 4, 2, 1):
                t = t + lax.gather(
                    t, (lanes ^ m)[:, None], dimension_numbers=dnums,
                    slice_sizes=(1,),
                    mode=lax.GatherScatterMode.PROMISE_IN_BOUNDS)
            a16 = 1.0 / (1.0 + jnp.exp(-t))
            for j in range(8):
                ybuf[r, pl.ds(16 * j, 16)] = xv[j] * a16

        pltpu.sync_copy(ybuf, acc.at[idxv], add=True)
        return 0

    lax.fori_loop(0, nchunks, chunk_body, 0)
    plsc.subcore_barrier()
    pltpu.sync_copy(acc.at[pl.ds(sid * 64, 64)],
                    out_hbm.at[pl.ds(cid * BP + sid * 64, 64)])


def _gate_tc(p_ref, cnt_ref, w_ref, c_ref):
    s = p_ref[0:BP, :] + p_ref[BP:2 * BP, :]
    mean = s / cnt_ref[...]
    c_ref[...] = jnp.tanh(jnp.dot(mean, w_ref[...],
                                  preferred_element_type=jnp.float32))


def _combine_tc(q_ref, o_ref):
    o_ref[...] = q_ref[0:B, :] + q_ref[BP:BP + B, :]


def kernel(x, W, batch, c_size):
    batch = batch.astype(jnp.int32)
    zeros = jnp.zeros((BP, D), jnp.float32)
    cnt = jnp.maximum(c_size, 1).astype(jnp.float32)
    cnt = jnp.concatenate([cnt, jnp.ones((BP - B,), jnp.float32)])[:, None]

    mesh = plsc.VectorSubcoreMesh(core_axis_name="c", subcore_axis_name="s")

    seg_partial = pl.kernel(
        _seg_sum_body,
        out_type=jax.ShapeDtypeStruct((NC * BP, D), jnp.float32),
        mesh=mesh,
        scratch_types=[
            pltpu.VMEM((CH, D), jnp.float32),
            pltpu.VMEM((CH,), jnp.int32),
            pltpu.VMEM_SHARED((BP, D), jnp.float32),
        ],
    )(x, batch, zeros)

    c = pl.pallas_call(
        _gate_tc,
        out_shape=jax.ShapeDtypeStruct((BP, D), jnp.float32),
    )(seg_partial, cnt, W)

    out_partial = pl.kernel(
        _gate_pool_body,
        out_type=jax.ShapeDtypeStruct((NC * BP, D), jnp.float32),
        mesh=mesh,
        scratch_types=[
            pltpu.VMEM((CH, D), jnp.float32),
            pltpu.VMEM((CH, D), jnp.float32),
            pltpu.VMEM((CH, D), jnp.float32),
            pltpu.VMEM((CH,), jnp.int32),
            pltpu.VMEM_SHARED((BP, D), jnp.float32),
            pltpu.VMEM_SHARED((BP, D), jnp.float32),
        ],
    )(x, batch, c, zeros)

    out = pl.pallas_call(
        _combine_tc,
        out_shape=jax.ShapeDtypeStruct((B, D), jnp.float32),
    )(out_partial)
    return out


# speedup vs baseline: 3.9287x; 1.4306x over previous
import jax
import jax.numpy as jnp
from jax import lax
from jax.experimental import pallas as pl
from jax.experimental.pallas import tpu as pltpu
from jax.experimental.pallas import tpu_sc as plsc

N = 320000
D = 128
B = 1000
BP = 1024
NC = 2
NS = 16
NW = NC * NS
CH = 128
NCHUNKS = N // CH
NBUF = 2

_LANES = None


def _start_fetch(x_hbm, b_hbm, xbuf, idxv, semx, semb, chunk, b):
    row0 = chunk * CH
    pltpu.make_async_copy(x_hbm.at[pl.ds(row0, CH)], xbuf.at[b],
                          semx.at[b]).start()
    pltpu.make_async_copy(b_hbm.at[pl.ds(row0, CH)], idxv.at[b],
                          semb.at[b]).start()


def _wait_fetch(x_hbm, b_hbm, xbuf, idxv, semx, semb, b):
    pltpu.make_async_copy(x_hbm.at[pl.ds(0, CH)], xbuf.at[b],
                          semx.at[b]).wait()
    pltpu.make_async_copy(b_hbm.at[pl.ds(0, CH)], idxv.at[b],
                          semb.at[b]).wait()


def _seg_sum_body(x_hbm, b_hbm, z_hbm, out_hbm, xbuf, idxv, acc, semx, semb):
    cid = lax.axis_index("c")
    sid = lax.axis_index("s")
    wid = sid * NC + cid

    pltpu.sync_copy(z_hbm.at[pl.ds(sid * 64, 64)], acc.at[pl.ds(sid * 64, 64)])
    plsc.subcore_barrier()

    nchunks = (NCHUNKS - wid + NW - 1) // NW

    for b in range(NBUF):
        @pl.when(b < nchunks)
        def _():
            _start_fetch(x_hbm, b_hbm, xbuf, idxv, semx, semb,
                         wid + b * NW, b)

    def chunk_body(k2, _):
        for b in range(NBUF):
            k = NBUF * k2 + b

            @pl.when(k < nchunks)
            def _():
                _wait_fetch(x_hbm, b_hbm, xbuf, idxv, semx, semb, b)
                pltpu.sync_copy(xbuf.at[b], acc.at[idxv.at[b]], add=True)

                @pl.when(k + NBUF < nchunks)
                def _():
                    _start_fetch(x_hbm, b_hbm, xbuf, idxv, semx, semb,
                                 wid + (k + NBUF) * NW, b)
        return 0

    lax.fori_loop(0, (nchunks + NBUF - 1) // NBUF, chunk_body, 0)
    plsc.subcore_barrier()
    pltpu.sync_copy(acc.at[pl.ds(sid * 64, 64)],
                    out_hbm.at[pl.ds(cid * BP + sid * 64, 64)])


def _gate_pool_body(x_hbm, b_hbm, c_hbm, z_hbm, out_hbm, xbuf, cbuf, ybuf,
                    idxv, acc, c_sh, semx, semb):
    cid = lax.axis_index("c")
    sid = lax.axis_index("s")
    wid = sid * NC + cid

    pltpu.sync_copy(z_hbm.at[pl.ds(sid * 64, 64)], acc.at[pl.ds(sid * 64, 64)])
    pltpu.sync_copy(c_hbm.at[pl.ds(sid * 64, 64)], c_sh.at[pl.ds(sid * 64, 64)])
    plsc.subcore_barrier()

    nchunks = (NCHUNKS - wid + NW - 1) // NW

    for b in range(NBUF):
        @pl.when(b < nchunks)
        def _():
            _start_fetch(x_hbm, b_hbm, xbuf, idxv, semx, semb,
                         wid + b * NW, b)

    lanes = lax.iota(jnp.int32, 16)
    dnums = lax.GatherDimensionNumbers(
        offset_dims=(), collapsed_slice_dims=(0,), start_index_map=(0,))

    def chunk_body(k2, _):
        for b in range(NBUF):
            k = NBUF * k2 + b

            @pl.when(k < nchunks)
            def _():
                _wait_fetch(x_hbm, b_hbm, xbuf, idxv, semx, semb, b)
                pltpu.sync_copy(c_sh.at[idxv.at[b]], cbuf)

                @plsc.parallel_loop(0, CH, 1, unroll=8)
                def row_body(r):
                    xv = [xbuf[b, r, pl.ds(16 * j, 16)] for j in range(8)]
                    cv = [cbuf[r, pl.ds(16 * j, 16)] for j in range(8)]
                    p = [xv[j] * cv[j] for j in range(8)]
                    p = [p[0] + p[1], p[2] + p[3], p[4] + p[5], p[6] + p[7]]
                    t = (p[0] + p[1]) + (p[2] + p[3])
                    for m in (8, 4, 2, 1):
                        t = t + lax.gather(
                            t, (lanes ^ m)[:, None], dimension_numbers=dnums,
                            slice_sizes=(1,),
                            mode=lax.GatherScatterMode.PROMISE_IN_BOUNDS)
                    a16 = 1.0 / (1.0 + jnp.exp(-t))
                    for j in range(8):
                        ybuf[r, pl.ds(16 * j, 16)] = xv[j] * a16

                pltpu.sync_copy(ybuf, acc.at[idxv.at[b]], add=True)

                @pl.when(k + NBUF < nchunks)
                def _():
                    _start_fetch(x_hbm, b_hbm, xbuf, idxv, semx, semb,
                                 wid + (k + NBUF) * NW, b)
        return 0

    lax.fori_loop(0, (nchunks + NBUF - 1) // NBUF, chunk_body, 0)
    plsc.subcore_barrier()
    pltpu.sync_copy(acc.at[pl.ds(sid * 64, 64)],
                    out_hbm.at[pl.ds(cid * BP + sid * 64, 64)])


def _gate_tc(p_ref, cnt_ref, w_ref, c_ref):
    s = p_ref[0:BP, :] + p_ref[BP:2 * BP, :]
    mean = s / cnt_ref[...]
    c_ref[...] = jnp.tanh(jnp.dot(mean, w_ref[...],
                                  preferred_element_type=jnp.float32))


def _combine_tc(q_ref, o_ref):
    o_ref[...] = q_ref[0:B, :] + q_ref[BP:BP + B, :]


def kernel(x, W, batch, c_size):
    batch = batch.astype(jnp.int32)
    zeros = jnp.zeros((BP, D), jnp.float32)
    cnt = jnp.maximum(c_size, 1).astype(jnp.float32)
    cnt = jnp.concatenate([cnt, jnp.ones((BP - B,), jnp.float32)])[:, None]

    mesh = plsc.VectorSubcoreMesh(core_axis_name="c", subcore_axis_name="s")

    seg_partial = pl.kernel(
        _seg_sum_body,
        out_type=jax.ShapeDtypeStruct((NC * BP, D), jnp.float32),
        mesh=mesh,
        scratch_types=[
            pltpu.VMEM((NBUF, CH, D), jnp.float32),
            pltpu.VMEM((NBUF, CH), jnp.int32),
            pltpu.VMEM_SHARED((BP, D), jnp.float32),
            pltpu.SemaphoreType.DMA((NBUF,)),
            pltpu.SemaphoreType.DMA((NBUF,)),
        ],
    )(x, batch, zeros)

    c = pl.pallas_call(
        _gate_tc,
        out_shape=jax.ShapeDtypeStruct((BP, D), jnp.float32),
    )(seg_partial, cnt, W)

    out_partial = pl.kernel(
        _gate_pool_body,
        out_type=jax.ShapeDtypeStruct((NC * BP, D), jnp.float32),
        mesh=mesh,
        scratch_types=[
            pltpu.VMEM((NBUF, CH, D), jnp.float32),
            pltpu.VMEM((CH, D), jnp.float32),
            pltpu.VMEM((CH, D), jnp.float32),
            pltpu.VMEM((NBUF, CH), jnp.int32),
            pltpu.VMEM_SHARED((BP, D), jnp.float32),
            pltpu.VMEM_SHARED((BP, D), jnp.float32),
            pltpu.SemaphoreType.DMA((NBUF,)),
            pltpu.SemaphoreType.DMA((NBUF,)),
        ],
    )(x, batch, c, zeros)

    out = pl.pallas_call(
        _combine_tc,
        out_shape=jax.ShapeDtypeStruct((B, D), jnp.float32),
    )(out_partial)
    return out


# speedup vs baseline: 4.5361x; 1.1546x over previous
import jax
import jax.numpy as jnp
from jax import lax
from jax.experimental import pallas as pl
from jax.experimental.pallas import tpu as pltpu
from jax.experimental.pallas import tpu_sc as plsc

N = 320000
D = 128
B = 1000
BP = 1024
NC = 2
NS = 16
NW = NC * NS
CH = 128
NCHUNKS = N // CH
NBUF = 2
NX = 3
NCB = 2


def _fetch(x_hbm, b_hbm, xbuf, idxv, semx, semb, chunk, b):
    row0 = chunk * CH
    return (pltpu.make_async_copy(x_hbm.at[pl.ds(row0, CH)], xbuf.at[b],
                                  semx.at[b]),
            pltpu.make_async_copy(b_hbm.at[pl.ds(row0, CH)], idxv.at[b],
                                  semb.at[b]))


def _start_fetch(x_hbm, b_hbm, xbuf, idxv, semx, semb, chunk, b):
    cx, cb = _fetch(x_hbm, b_hbm, xbuf, idxv, semx, semb, chunk, b)
    cx.start()
    cb.start()


def _wait_fetch(x_hbm, b_hbm, xbuf, idxv, semx, semb, b):
    cx, cb = _fetch(x_hbm, b_hbm, xbuf, idxv, semx, semb, 0, b)
    cx.wait()
    cb.wait()


def _seg_sum_body(x_hbm, b_hbm, z_hbm, out_hbm, xbuf, idxv, acc, semx, semb):
    cid = lax.axis_index("c")
    sid = lax.axis_index("s")
    wid = sid * NC + cid

    pltpu.sync_copy(z_hbm.at[pl.ds(sid * 64, 64)], acc.at[pl.ds(sid * 64, 64)])
    plsc.subcore_barrier()

    nchunks = (NCHUNKS - wid + NW - 1) // NW

    for b in range(NBUF):
        @pl.when(b < nchunks)
        def _():
            _start_fetch(x_hbm, b_hbm, xbuf, idxv, semx, semb,
                         wid + b * NW, b)

    def chunk_body(k2, _):
        for b in range(NBUF):
            k = NBUF * k2 + b

            @pl.when(k < nchunks)
            def _():
                _wait_fetch(x_hbm, b_hbm, xbuf, idxv, semx, semb, b)
                pltpu.sync_copy(xbuf.at[b], acc.at[idxv.at[b]], add=True)

                @pl.when(k + NBUF < nchunks)
                def _():
                    _start_fetch(x_hbm, b_hbm, xbuf, idxv, semx, semb,
                                 wid + (k + NBUF) * NW, b)
        return 0

    lax.fori_loop(0, (nchunks + NBUF - 1) // NBUF, chunk_body, 0)
    plsc.subcore_barrier()
    pltpu.sync_copy(acc.at[pl.ds(sid * 64, 64)],
                    out_hbm.at[pl.ds(cid * BP + sid * 64, 64)])


def _gate_pool_body(x_hbm, b_hbm, c_hbm, z_hbm, out_hbm, xbuf, cbuf, ybuf,
                    idxv, acc, c_sh, semx, semb, semg):
    cid = lax.axis_index("c")
    sid = lax.axis_index("s")
    wid = sid * NC + cid

    pltpu.sync_copy(z_hbm.at[pl.ds(sid * 64, 64)], acc.at[pl.ds(sid * 64, 64)])
    pltpu.sync_copy(c_hbm.at[pl.ds(sid * 64, 64)], c_sh.at[pl.ds(sid * 64, 64)])
    plsc.subcore_barrier()

    n = (NCHUNKS - wid + NW - 1) // NW

    for b in range(NBUF):
        @pl.when(b < n)
        def _():
            _start_fetch(x_hbm, b_hbm, xbuf, idxv, semx, semb,
                         wid + b * NW, b)

    lanes = lax.iota(jnp.int32, 16)
    dnums = lax.GatherDimensionNumbers(
        offset_dims=(), collapsed_slice_dims=(0,), start_index_map=(0,))

    def pipe_body(i2, _):
        for b6 in range(6):
            i = 6 * i2 + b6
            xs = b6 % NX
            cs = b6 % NCB

            @pl.when(i < n)
            def _():
                _wait_fetch(x_hbm, b_hbm, xbuf, idxv, semx, semb, xs)
                pltpu.make_async_copy(c_sh.at[idxv.at[xs]], cbuf.at[cs],
                                      semg.at[cs]).start()

            @pl.when((i >= 1) & (i <= n))
            def _():
                js = (b6 - 1) % NX
                jc = (b6 - 1) % NCB
                pltpu.make_async_copy(c_sh.at[idxv.at[js]], cbuf.at[jc],
                                      semg.at[jc]).wait()

                @plsc.parallel_loop(0, CH, 1, unroll=8)
                def row_body(r):
                    xv = [xbuf[js, r, pl.ds(16 * j, 16)] for j in range(8)]
                    cv = [cbuf[jc, r, pl.ds(16 * j, 16)] for j in range(8)]
                    p = [xv[j] * cv[j] for j in range(8)]
                    p = [p[0] + p[1], p[2] + p[3], p[4] + p[5], p[6] + p[7]]
                    t = (p[0] + p[1]) + (p[2] + p[3])
                    for m in (8, 4, 2, 1):
                        t = t + lax.gather(
                            t, (lanes ^ m)[:, None], dimension_numbers=dnums,
                            slice_sizes=(1,),
                            mode=lax.GatherScatterMode.PROMISE_IN_BOUNDS)
                    a16 = 1.0 / (1.0 + jnp.exp(-t))
                    for j in range(8):
                        ybuf[r, pl.ds(16 * j, 16)] = xv[j] * a16

                pltpu.sync_copy(ybuf, acc.at[idxv.at[js]], add=True)

            @pl.when(i + NBUF < n)
            def _():
                _start_fetch(x_hbm, b_hbm, xbuf, idxv, semx, semb,
                             wid + (i + NBUF) * NW, (b6 + NBUF) % NX)
        return 0

    lax.fori_loop(0, (n + 1 + 5) // 6, pipe_body, 0)
    plsc.subcore_barrier()
    pltpu.sync_copy(acc.at[pl.ds(sid * 64, 64)],
                    out_hbm.at[pl.ds(cid * BP + sid * 64, 64)])


def _gate_tc(p_ref, cnt_ref, w_ref, c_ref):
    s = p_ref[0:BP, :] + p_ref[BP:2 * BP, :]
    mean = s / cnt_ref[...]
    c_ref[...] = jnp.tanh(jnp.dot(mean, w_ref[...],
                                  preferred_element_type=jnp.float32))


def _combine_tc(q_ref, o_ref):
    o_ref[...] = q_ref[0:B, :] + q_ref[BP:BP + B, :]


def kernel(x, W, batch, c_size):
    batch = batch.astype(jnp.int32)
    zeros = jnp.zeros((BP, D), jnp.float32)
    cnt = jnp.maximum(c_size, 1).astype(jnp.float32)
    cnt = jnp.concatenate([cnt, jnp.ones((BP - B,), jnp.float32)])[:, None]

    mesh = plsc.VectorSubcoreMesh(core_axis_name="c", subcore_axis_name="s")

    seg_partial = pl.kernel(
        _seg_sum_body,
        out_type=jax.ShapeDtypeStruct((NC * BP, D), jnp.float32),
        mesh=mesh,
        scratch_types=[
            pltpu.VMEM((NBUF, CH, D), jnp.float32),
            pltpu.VMEM((NBUF, CH), jnp.int32),
            pltpu.VMEM_SHARED((BP, D), jnp.float32),
            pltpu.SemaphoreType.DMA((NBUF,)),
            pltpu.SemaphoreType.DMA((NBUF,)),
        ],
    )(x, batch, zeros)

    c = pl.pallas_call(
        _gate_tc,
        out_shape=jax.ShapeDtypeStruct((BP, D), jnp.float32),
    )(seg_partial, cnt, W)

    out_partial = pl.kernel(
        _gate_pool_body,
        out_type=jax.ShapeDtypeStruct((NC * BP, D), jnp.float32),
        mesh=mesh,
        scratch_types=[
            pltpu.VMEM((NX, CH, D), jnp.float32),
            pltpu.VMEM((NCB, CH, D), jnp.float32),
            pltpu.VMEM((CH, D), jnp.float32),
            pltpu.VMEM((NX, CH), jnp.int32),
            pltpu.VMEM_SHARED((BP, D), jnp.float32),
            pltpu.VMEM_SHARED((BP, D), jnp.float32),
            pltpu.SemaphoreType.DMA((NX,)),
            pltpu.SemaphoreType.DMA((NX,)),
            pltpu.SemaphoreType.DMA((NCB,)),
        ],
    )(x, batch, c, zeros)

    out = pl.pallas_call(
        _combine_tc,
        out_shape=jax.ShapeDtypeStruct((B, D), jnp.float32),
    )(out_partial)
    return out


# speedup vs baseline: 5.3961x; 1.1896x over previous
import jax
import jax.numpy as jnp
from jax import lax
from jax.experimental import pallas as pl
from jax.experimental.pallas import tpu as pltpu
from jax.experimental.pallas import tpu_sc as plsc

N = 320000
D = 128
B = 1000
BP = 1024
NC = 2
NS = 16
NW = NC * NS
CH = 128
NCHUNKS = N // CH
NBUF = 2
NX = 3
NCB = 2


def _fetch(x_hbm, b_hbm, xbuf, idxv, semx, semb, chunk, b):
    row0 = chunk * CH
    return (pltpu.make_async_copy(x_hbm.at[pl.ds(row0, CH)], xbuf.at[b],
                                  semx.at[b]),
            pltpu.make_async_copy(b_hbm.at[pl.ds(row0, CH)], idxv.at[b],
                                  semb.at[b]))


def _start_fetch(x_hbm, b_hbm, xbuf, idxv, semx, semb, chunk, b):
    cx, cb = _fetch(x_hbm, b_hbm, xbuf, idxv, semx, semb, chunk, b)
    cx.start()
    cb.start()


def _wait_fetch(x_hbm, b_hbm, xbuf, idxv, semx, semb, b):
    cx, cb = _fetch(x_hbm, b_hbm, xbuf, idxv, semx, semb, 0, b)
    cx.wait()
    cb.wait()


def _seg_sum_body(x_hbm, b_hbm, z_hbm, out_hbm, xbuf, idxv, acc, semx, semb):
    cid = lax.axis_index("c")
    sid = lax.axis_index("s")
    wid = sid * NC + cid

    pltpu.sync_copy(z_hbm.at[pl.ds(sid * 64, 64)], acc.at[pl.ds(sid * 64, 64)])
    plsc.subcore_barrier()

    nchunks = (NCHUNKS - wid + NW - 1) // NW

    for b in range(NBUF):
        @pl.when(b < nchunks)
        def _():
            _start_fetch(x_hbm, b_hbm, xbuf, idxv, semx, semb,
                         wid + b * NW, b)

    def chunk_body(k2, _):
        for b in range(NBUF):
            k = NBUF * k2 + b

            @pl.when(k < nchunks)
            def _():
                _wait_fetch(x_hbm, b_hbm, xbuf, idxv, semx, semb, b)
                pltpu.sync_copy(xbuf.at[b], acc.at[idxv.at[b]], add=True)

                @pl.when(k + NBUF < nchunks)
                def _():
                    _start_fetch(x_hbm, b_hbm, xbuf, idxv, semx, semb,
                                 wid + (k + NBUF) * NW, b)
        return 0

    lax.fori_loop(0, (nchunks + NBUF - 1) // NBUF, chunk_body, 0)
    plsc.subcore_barrier()
    pltpu.sync_copy(acc.at[pl.ds(sid * 64, 64)],
                    out_hbm.at[pl.ds(cid * BP + sid * 64, 64)])


def _gate_pool_body(x_hbm, b_hbm, c_hbm, z_hbm, out_hbm, xbuf, cbuf,
                    idxv, acc, c_sh, semx, semb, semg, sems):
    cid = lax.axis_index("c")
    sid = lax.axis_index("s")
    wid = sid * NC + cid

    pltpu.sync_copy(z_hbm.at[pl.ds(sid * 64, 64)], acc.at[pl.ds(sid * 64, 64)])
    pltpu.sync_copy(c_hbm.at[pl.ds(sid * 64, 64)], c_sh.at[pl.ds(sid * 64, 64)])
    plsc.subcore_barrier()

    n = (NCHUNKS - wid + NW - 1) // NW

    for b in range(NBUF):
        @pl.when(b < n)
        def _():
            _start_fetch(x_hbm, b_hbm, xbuf, idxv, semx, semb,
                         wid + b * NW, b)

    lanes = lax.iota(jnp.int32, 16)
    dnums = lax.GatherDimensionNumbers(
        offset_dims=(), collapsed_slice_dims=(0,), start_index_map=(0,))

    def pipe_body(i2, _):
        for b4 in range(4):
            i = 4 * i2 + b4
            xs = b4 % 4
            cs = b4 % 2

            @pl.when(i < n)
            def _():
                _wait_fetch(x_hbm, b_hbm, xbuf, idxv, semx, semb, xs)
                pltpu.make_async_copy(c_sh.at[idxv.at[xs]], cbuf.at[cs],
                                      semg.at[cs]).start()

            @pl.when((i >= 1) & (i <= n))
            def _():
                js = (b4 - 1) % 4
                jc = (b4 - 1) % 2
                pltpu.make_async_copy(c_sh.at[idxv.at[js]], cbuf.at[jc],
                                      semg.at[jc]).wait()

                @plsc.parallel_loop(0, CH, 1, unroll=8)
                def row_body(r):
                    xv = [xbuf[js, r, pl.ds(16 * j, 16)] for j in range(8)]
                    cv = [cbuf[jc, r, pl.ds(16 * j, 16)] for j in range(8)]
                    p = [xv[j] * cv[j] for j in range(8)]
                    p = [p[0] + p[1], p[2] + p[3], p[4] + p[5], p[6] + p[7]]
                    t = (p[0] + p[1]) + (p[2] + p[3])
                    for m in (8, 4, 2, 1):
                        t = t + lax.gather(
                            t, (lanes ^ m)[:, None], dimension_numbers=dnums,
                            slice_sizes=(1,),
                            mode=lax.GatherScatterMode.PROMISE_IN_BOUNDS)
                    a16 = 1.0 / (1.0 + jnp.exp(-t))
                    for j in range(8):
                        xbuf[js, r, pl.ds(16 * j, 16)] = xv[j] * a16

                pltpu.async_copy(xbuf.at[js], acc.at[idxv.at[js]],
                                 sems.at[jc], add=True)

            @pl.when((i >= 2) & (i - 2 < n))
            def _():
                ds2 = (b4 - 2) % 4
                pltpu.make_async_copy(xbuf.at[ds2], acc.at[idxv.at[ds2]],
                                      sems.at[b4 % 2]).wait()

            @pl.when(i + NBUF < n)
            def _():
                _start_fetch(x_hbm, b_hbm, xbuf, idxv, semx, semb,
                             wid + (i + NBUF) * NW, (b4 + NBUF) % 4)
        return 0

    lax.fori_loop(0, (n + 2 + 3) // 4, pipe_body, 0)
    plsc.subcore_barrier()
    pltpu.sync_copy(acc.at[pl.ds(sid * 64, 64)],
                    out_hbm.at[pl.ds(cid * BP + sid * 64, 64)])


def _gate_tc(p_ref, cnt_ref, w_ref, c_ref):
    s = p_ref[0:BP, :] + p_ref[BP:2 * BP, :]
    mean = s / cnt_ref[...]
    c_ref[...] = jnp.tanh(jnp.dot(mean, w_ref[...],
                                  preferred_element_type=jnp.float32))


def _combine_tc(q_ref, o_ref):
    o_ref[...] = q_ref[0:B, :] + q_ref[BP:BP + B, :]


def kernel(x, W, batch, c_size):
    batch = batch.astype(jnp.int32)
    zeros = jnp.zeros((BP, D), jnp.float32)
    cnt = jnp.maximum(c_size, 1).astype(jnp.float32)
    cnt = jnp.concatenate([cnt, jnp.ones((BP - B,), jnp.float32)])[:, None]

    mesh = plsc.VectorSubcoreMesh(core_axis_name="c", subcore_axis_name="s")

    seg_partial = pl.kernel(
        _seg_sum_body,
        out_type=jax.ShapeDtypeStruct((NC * BP, D), jnp.float32),
        mesh=mesh,
        scratch_types=[
            pltpu.VMEM((NBUF, CH, D), jnp.float32),
            pltpu.VMEM((NBUF, CH), jnp.int32),
            pltpu.VMEM_SHARED((BP, D), jnp.float32),
            pltpu.SemaphoreType.DMA((NBUF,)),
            pltpu.SemaphoreType.DMA((NBUF,)),
        ],
    )(x, batch, zeros)

    c = pl.pallas_call(
        _gate_tc,
        out_shape=jax.ShapeDtypeStruct((BP, D), jnp.float32),
    )(seg_partial, cnt, W)

    out_partial = pl.kernel(
        _gate_pool_body,
        out_type=jax.ShapeDtypeStruct((NC * BP, D), jnp.float32),
        mesh=mesh,
        scratch_types=[
            pltpu.VMEM((4, CH, D), jnp.float32),
            pltpu.VMEM((2, CH, D), jnp.float32),
            pltpu.VMEM((4, CH), jnp.int32),
            pltpu.VMEM_SHARED((BP, D), jnp.float32),
            pltpu.VMEM_SHARED((BP, D), jnp.float32),
            pltpu.SemaphoreType.DMA((4,)),
            pltpu.SemaphoreType.DMA((4,)),
            pltpu.SemaphoreType.DMA((2,)),
            pltpu.SemaphoreType.DMA((2,)),
        ],
    )(x, batch, c, zeros)

    out = pl.pallas_call(
        _combine_tc,
        out_shape=jax.ShapeDtypeStruct((B, D), jnp.float32),
    )(out_partial)
    return out
